# baseline (speedup 1.0000x reference)
import functools

import jax
import jax.numpy as jnp
from jax import lax
from jax.experimental import pallas as pl
from jax.experimental.pallas import tpu as pltpu
from jax.experimental.pallas import tpu_sc as plsc

NUM_VARS = 26
VOCAB = 100000
D_MODEL = 32
NUM_PATCHES = 16
BATCH = 4096

NUM_CORES = 2
NUM_SUBCORES = 16
NUM_WORKERS = NUM_CORES * NUM_SUBCORES
BB = 128
LANES = 16
NGRP = BB // LANES
D8 = D_MODEL // 8
B128 = BATCH // 128


def _gather_body(idx_hbm, tab_hbm, cout_hbm, gidall, grows, tcol,
                 isem, gsem, wsem):
    wid = lax.axis_index("s") * NUM_CORES + lax.axis_index("c")
    b0 = wid * BB

    pltpu.async_copy(idx_hbm.at[:, pl.ds(b0, BB)], gidall, isem).wait()
    for v in range(NUM_VARS):
        for g in range(NGRP):
            sl = pl.ds(g * LANES, LANES)
            gidall[v, sl] = gidall[v, sl] + v * VOCAB

    lane = lax.broadcasted_iota(jnp.int32, (LANES,), 0)
    zero = lane - lane

    for v in range(NUM_VARS):
        for g in range(NGRP):
            idxv = gidall[v, pl.ds(g * LANES, LANES)]
            pltpu.async_copy(
                tab_hbm.at[idxv], grows.at[v, pl.ds(g * LANES, LANES)], gsem
            )

    def extract(v, t):
        vsplat = zero + v
        for d8 in range(D8):
            for dd in range(8):
                colv = zero + (d8 * 8 + dd)
                for g in range(NGRP):
                    tcol[t, d8, dd, pl.ds(g * LANES, LANES)] = (
                        plsc.load_gather(
                            grows, [vsplat, lane + g * LANES, colv]
                        )
                    )

    def cwrite_drain():
        pltpu.make_async_copy(
            tcol.at[0], cout_hbm.at[0, :, 0], wsem
        ).wait()

    def body(j, _):
        for t in (0, 1):
            v = 2 * j + t
            for _ in range(NGRP):
                pltpu.make_async_copy(
                    tab_hbm.at[gidall[0, pl.ds(0, LANES)]],
                    grows.at[0, pl.ds(0, LANES)],
                    gsem,
                ).wait()

            @pl.when(v >= 2)
            def _():
                cwrite_drain()

            extract(v, t)
            pltpu.async_copy(tcol.at[t], cout_hbm.at[v, :, wid], wsem)
        return 0

    lax.fori_loop(0, NUM_VARS // 2, body, 0)
    cwrite_drain()
    cwrite_drain()


def _repeat_body(cin_hbm, out_hbm, shbuf, lsem):
    c = lax.axis_index("c")
    s = lax.axis_index("s")
    for i in range(NUM_VARS // NUM_CORES):
        v = 2 * i + c

        @pl.when(s == 0)
        def _():
            pltpu.async_copy(cin_hbm.at[v], shbuf, lsem).wait()

        plsc.subcore_barrier()
        pltpu.sync_copy(shbuf, out_hbm.at[v, s])
        plsc.subcore_barrier()


@jax.jit
def _emb_call(idx_t, tab_flat):
    mesh = plsc.VectorSubcoreMesh(core_axis_name="c", subcore_axis_name="s")
    compact = pl.kernel(
        _gather_body,
        out_type=jax.ShapeDtypeStruct(
            (NUM_VARS, D8, B128, 8, 128), jnp.float32
        ),
        mesh=mesh,
        compiler_params=pltpu.CompilerParams(
            use_tc_tiling_on_sc=False, needs_layout_passes=False
        ),
        scratch_types=[
            pltpu.VMEM((NUM_VARS, BB), jnp.int32),
            pltpu.VMEM((NUM_VARS, BB, D_MODEL), jnp.float32),
            pltpu.VMEM((2, D8, 8, BB), jnp.float32),
            pltpu.SemaphoreType.DMA,
            pltpu.SemaphoreType.DMA,
            pltpu.SemaphoreType.DMA,
        ],
    )(idx_t, tab_flat)
    return pl.kernel(
        _repeat_body,
        out_type=jax.ShapeDtypeStruct(
            (NUM_VARS, NUM_PATCHES, D8, B128, 8, 128), jnp.float32
        ),
        mesh=mesh,
        compiler_params=pltpu.CompilerParams(use_tc_tiling_on_sc=False),
        scratch_types=[
            pltpu.VMEM_SHARED((D8, B128, 8, 128), jnp.float32),
            pltpu.SemaphoreType.DMA,
        ],
    )(compact)


def kernel(static_categorical_values, tables):
    idx_t = jnp.transpose(static_categorical_values.astype(jnp.int32))
    tab_flat = tables.reshape(NUM_VARS * VOCAB, D_MODEL)
    out6 = _emb_call(idx_t, tab_flat)
    return (
        out6.transpose(3, 5, 0, 1, 2, 4)
        .reshape(BATCH, NUM_VARS, NUM_PATCHES, D_MODEL)
    )


# speedup vs baseline: 1.0008x; 1.0008x over previous
import functools

import jax
import jax.numpy as jnp
from jax import lax
from jax.experimental import pallas as pl
from jax.experimental.pallas import tpu as pltpu
from jax.experimental.pallas import tpu_sc as plsc

NUM_VARS = 26
VOCAB = 100000
D_MODEL = 32
NUM_PATCHES = 16
BATCH = 4096

NUM_CORES = 2
NUM_SUBCORES = 16
NUM_WORKERS = NUM_CORES * NUM_SUBCORES
BB = 128
LANES = 16
NGRP = BB // LANES
D8 = D_MODEL // 8
B128 = BATCH // 128


def _gather_body(idx_hbm, tab_hbm, cout_hbm, gidall, grows, tcol,
                 isem, gsem0, gsem1, gsem2, gsem3, wsem):
    gsems = (gsem0, gsem1, gsem2, gsem3)
    wid = lax.axis_index("s") * NUM_CORES + lax.axis_index("c")
    b0 = wid * BB

    pltpu.async_copy(idx_hbm.at[:, pl.ds(b0, BB)], gidall, isem).wait()
    for v in range(NUM_VARS):
        for g in range(NGRP):
            sl = pl.ds(g * LANES, LANES)
            gidall[v, sl] = gidall[v, sl] + v * VOCAB

    lane = lax.broadcasted_iota(jnp.int32, (LANES,), 0)
    zero = lane - lane

    for v in range(NUM_VARS):
        for g in range(NGRP):
            idxv = gidall[v, pl.ds(g * LANES, LANES)]
            pltpu.async_copy(
                tab_hbm.at[idxv],
                grows.at[v, pl.ds(g * LANES, LANES)],
                gsems[(v * NGRP + g) % 4],
            )

    def extract(v, t):
        vsplat = zero + v
        for d8 in range(D8):
            for dd in range(8):
                colv = zero + (d8 * 8 + dd)
                for g in range(NGRP):
                    tcol[t, d8, dd, pl.ds(g * LANES, LANES)] = (
                        plsc.load_gather(
                            grows, [vsplat, lane + g * LANES, colv]
                        )
                    )

    def cwrite_drain():
        pltpu.make_async_copy(
            tcol.at[0], cout_hbm.at[0, :, 0], wsem
        ).wait()

    def body(j, _):
        for t in (0, 1):
            v = 2 * j + t
            for g in range(NGRP):
                pltpu.make_async_copy(
                    tab_hbm.at[gidall[0, pl.ds(0, LANES)]],
                    grows.at[0, pl.ds(0, LANES)],
                    gsems[g % 4],
                ).wait()

            @pl.when(v >= 2)
            def _():
                cwrite_drain()

            extract(v, t)
            pltpu.async_copy(tcol.at[t], cout_hbm.at[v, :, wid], wsem)
        return 0

    lax.fori_loop(0, NUM_VARS // 2, body, 0)
    cwrite_drain()
    cwrite_drain()


def _repeat_body(cin_hbm, out_hbm, shbuf, lsem):
    c = lax.axis_index("c")
    s = lax.axis_index("s")
    for i in range(NUM_VARS // NUM_CORES):
        v = 2 * i + c

        @pl.when(s == 0)
        def _():
            pltpu.async_copy(cin_hbm.at[v], shbuf, lsem).wait()

        plsc.subcore_barrier()
        pltpu.sync_copy(shbuf, out_hbm.at[v, s])
        plsc.subcore_barrier()


@jax.jit
def _emb_call(idx_t, tab_flat):
    mesh = plsc.VectorSubcoreMesh(core_axis_name="c", subcore_axis_name="s")
    compact = pl.kernel(
        _gather_body,
        out_type=jax.ShapeDtypeStruct(
            (NUM_VARS, D8, B128, 8, 128), jnp.float32
        ),
        mesh=mesh,
        compiler_params=pltpu.CompilerParams(
            use_tc_tiling_on_sc=False, needs_layout_passes=False
        ),
        scratch_types=[
            pltpu.VMEM((NUM_VARS, BB), jnp.int32),
            pltpu.VMEM((NUM_VARS, BB, D_MODEL), jnp.float32),
            pltpu.VMEM((2, D8, 8, BB), jnp.float32),
            pltpu.SemaphoreType.DMA,
            pltpu.SemaphoreType.DMA,
            pltpu.SemaphoreType.DMA,
            pltpu.SemaphoreType.DMA,
            pltpu.SemaphoreType.DMA,
            pltpu.SemaphoreType.DMA,
        ],
    )(idx_t, tab_flat)
    return pl.kernel(
        _repeat_body,
        out_type=jax.ShapeDtypeStruct(
            (NUM_VARS, NUM_PATCHES, D8, B128, 8, 128), jnp.float32
        ),
        mesh=mesh,
        compiler_params=pltpu.CompilerParams(use_tc_tiling_on_sc=False),
        scratch_types=[
            pltpu.VMEM_SHARED((D8, B128, 8, 128), jnp.float32),
            pltpu.SemaphoreType.DMA,
        ],
    )(compact)


def kernel(static_categorical_values, tables):
    idx_t = jnp.transpose(static_categorical_values.astype(jnp.int32))
    tab_flat = tables.reshape(NUM_VARS * VOCAB, D_MODEL)
    out6 = _emb_call(idx_t, tab_flat)
    return (
        out6.transpose(3, 5, 0, 1, 2, 4)
        .reshape(BATCH, NUM_VARS, NUM_PATCHES, D_MODEL)
    )


# speedup vs baseline: 1.0783x; 1.0774x over previous
import functools

import jax
import jax.numpy as jnp
from jax import lax
from jax.experimental import pallas as pl
from jax.experimental.pallas import tpu as pltpu
from jax.experimental.pallas import tpu_sc as plsc

NUM_VARS = 26
VOCAB = 100000
D_MODEL = 32
NUM_PATCHES = 16
BATCH = 4096

NUM_CORES = 2
NUM_SUBCORES = 16
NUM_WORKERS = NUM_CORES * NUM_SUBCORES
BB = 128
LANES = 16
NGRP = BB // LANES
PACK = 128 // D_MODEL
QROWS = NUM_VARS * VOCAB // PACK


def _emb_body(idx_hbm, tab_hbm, out_hbm, idxt, qbufall, qoffall, grows, tcol,
              isem, gsem, wsem):
    wid = lax.axis_index("s") * NUM_CORES + lax.axis_index("c")
    b0 = wid * BB
    lane = lax.broadcasted_iota(jnp.int32, (LANES,), 0)

    def gather_start(v, t):
        return pltpu.async_copy(
            tab_hbm.at[qbufall.at[lax.rem(v, 8)]], grows.at[t], gsem
        )

    def gather_wait(t):
        pltpu.make_async_copy(
            tab_hbm.at[qbufall.at[0]], grows.at[t], gsem
        ).wait()

    def drain_writes():
        for _ in range(NUM_PATCHES):
            pltpu.make_async_copy(
                tcol.at[0], out_hbm.at[0, 0, :, pl.ds(0, BB)], wsem
            ).wait()

    def extract(v, t):
        base = lax.rem(v, 8) * BB
        for d in range(D_MODEL):
            for g in range(NGRP):
                offv = qoffall[pl.ds(base + g * LANES, LANES)]
                tcol[t, d, pl.ds(g * LANES, LANES)] = plsc.load_gather(
                    grows.at[t], [lane + g * LANES, offv + d]
                )

    def body(j, _):
        for t in (0, 1):
            v = 2 * j + t
            if t == 0:
                @pl.when(lax.rem(v, 8) == 0)
                def _():
                    pltpu.async_copy(
                        idx_hbm.at[
                            pl.ds(pl.multiple_of(v, 8), 8), pl.ds(b0, BB)
                        ],
                        idxt,
                        isem,
                    ).wait()
                    for r in range(8):
                        for g in range(NGRP):
                            sl = pl.ds(g * LANES, LANES)
                            gidv = idxt[r, sl] + (v + r) * VOCAB
                            qbufall[r, sl] = lax.shift_right_logical(gidv, 2)
                            qoffall[pl.ds(r * BB + g * LANES, LANES)] = (
                                lax.shift_left(lax.bitwise_and(gidv, 3), 5)
                            )
                    gather_start(v, 0)

            gather_wait(t)

            @pl.when(
                jnp.logical_and(v + 1 < NUM_VARS, lax.rem(v + 1, 8) != 0)
            )
            def _():
                gather_start(v + 1, 1 - t)

            @pl.when(v >= 2)
            def _():
                drain_writes()

            extract(v, t)
            for p in range(NUM_PATCHES):
                pltpu.async_copy(
                    tcol.at[t], out_hbm.at[v, p, :, pl.ds(b0, BB)], wsem
                )
        return 0

    lax.fori_loop(0, NUM_VARS // 2, body, 0)
    drain_writes()
    drain_writes()


@jax.jit
def _emb_call(idx_t, tab_q):
    mesh = plsc.VectorSubcoreMesh(core_axis_name="c", subcore_axis_name="s")
    return pl.kernel(
        _emb_body,
        out_type=jax.ShapeDtypeStruct(
            (NUM_VARS, NUM_PATCHES, D_MODEL, BATCH), jnp.float32
        ),
        mesh=mesh,
        compiler_params=pltpu.CompilerParams(
            use_tc_tiling_on_sc=True, needs_layout_passes=False
        ),
        scratch_types=[
            pltpu.VMEM((8, BB), jnp.int32),
            pltpu.VMEM((8, BB), jnp.int32),
            pltpu.VMEM((8 * BB,), jnp.int32),
            pltpu.VMEM((2, BB, 128), jnp.float32),
            pltpu.VMEM((2, D_MODEL, BB), jnp.float32),
            pltpu.SemaphoreType.DMA,
            pltpu.SemaphoreType.DMA,
            pltpu.SemaphoreType.DMA,
        ],
    )(idx_t, tab_q)


def kernel(static_categorical_values, tables):
    idx_t = jnp.transpose(static_categorical_values.astype(jnp.int32))
    tab_q = tables.reshape(QROWS, 128)
    out = _emb_call(idx_t, tab_q)
    return jnp.transpose(out, (3, 0, 1, 2))
